# baseline (speedup 1.0000x reference)
import jax
import jax.numpy as jnp
from jax.experimental import pallas as pl
from jax.experimental.pallas import tpu as pltpu


def _mm_body(adj_ref, w_ref, out_ref):
    j = pl.program_id(1)
    part = jnp.dot(adj_ref[...], w_ref[...], preferred_element_type=jnp.float32)

    @pl.when(j == 0)
    def _():
        out_ref[...] = part

    @pl.when(j == 1)
    def _():
        out_ref[...] += part


def kernel(adj, weight):
    m, k = adj.shape
    k2, n = weight.shape
    assert k == k2
    bm = 256
    bk = k // 2
    grid = (m // bm, 2)
    return pl.pallas_call(
        _mm_body,
        grid=grid,
        in_specs=[
            pl.BlockSpec((bm, bk), lambda i, j: (i, j)),
            pl.BlockSpec((bk, n), lambda i, j: (j, 0)),
        ],
        out_specs=pl.BlockSpec((bm, n), lambda i, j: (i, 0)),
        out_shape=jax.ShapeDtypeStruct((m, n), jnp.float32),
        compiler_params=pltpu.CompilerParams(
            dimension_semantics=("arbitrary", "arbitrary"),
        ),
    )(adj, weight)
